# Initial kernel scaffold; baseline (speedup 1.0000x reference)
import jax
import jax.numpy as jnp
from jax.experimental import pallas as pl


def kernel(node_feats, edge_feats, edge_index, W_sg, b_sg, W_dg, b_dg, W_eg, b_eg, W_su, b_su, W_du, b_du, ln_n_w, ln_n_b, ln_e_w, ln_e_b):
    raise NotImplementedError("write your pallas kernel here")


# speedup vs baseline: 2.7333x; 2.7333x over previous
import functools

import jax
import jax.numpy as jnp
from jax import lax
from jax.experimental import pallas as pl
from jax.experimental.pallas import tpu as pltpu
from jax.experimental.pallas import tpu_sc as plsc

N = 10000
E = 320000
D = 128

NC = 2
NS = 16
NW = NC * NS

EPW = E // NW
KG = 80
_G_STEPS = EPW // KG
assert _G_STEPS * KG == EPW

@functools.cache
def _build_gather():
    mesh = plsc.VectorSubcoreMesh(core_axis_name="c", subcore_axis_name="s",
                                  num_cores=NC, num_subcores=NS)

    @functools.partial(
        pl.kernel,
        out_type=(
            jax.ShapeDtypeStruct((E, D), jnp.float32),
            jax.ShapeDtypeStruct((E, D), jnp.float32),
        ),
        mesh=mesh,
        scratch_types=[
            pltpu.VMEM((KG,), jnp.int32),
            pltpu.VMEM((KG,), jnp.int32),
            pltpu.VMEM((KG, D), jnp.float32),
            pltpu.VMEM((KG, D), jnp.float32),
            pltpu.SemaphoreType.DMA,
            pltpu.SemaphoreType.DMA,
        ],
    )
    def _gather_nodes(nf_hbm, src_hbm, dst_hbm, osrc_hbm, odst_hbm,
                      idx_s, idx_d, rows_s, rows_d, sem_s, sem_d):
        wid = lax.axis_index("s") * NC + lax.axis_index("c")
        base = wid * EPW

        def step(i, _):
            off = base + i * KG
            pltpu.sync_copy(src_hbm.at[pl.ds(off, KG)], idx_s)
            pltpu.sync_copy(dst_hbm.at[pl.ds(off, KG)], idx_d)
            a = pltpu.async_copy(nf_hbm.at[idx_s], rows_s, sem_s)
            b = pltpu.async_copy(nf_hbm.at[idx_d], rows_d, sem_d)
            a.wait()
            b.wait()
            pltpu.sync_copy(rows_s, osrc_hbm.at[pl.ds(off, KG)])
            pltpu.sync_copy(rows_d, odst_hbm.at[pl.ds(off, KG)])
            return 0

        lax.fori_loop(0, _G_STEPS, step, 0)

    return _gather_nodes


EPT = E // NS
KS = 80
_S_STEPS = EPT // KS
assert _S_STEPS * KS == EPT
NPT = (N // NS) // 8 * 8
NTAIL = N - NS * NPT


@functools.cache
def _build_scatter():
    mesh = plsc.VectorSubcoreMesh(core_axis_name="c", subcore_axis_name="s",
                                  num_cores=NC, num_subcores=NS)

    @functools.partial(
        pl.kernel,
        out_type=(
            jax.ShapeDtypeStruct((N, D), jnp.float32),
            jax.ShapeDtypeStruct((N, D), jnp.float32),
        ),
        mesh=mesh,
        scratch_types=[
            pltpu.VMEM((KS,), jnp.int32),
            pltpu.VMEM((KS, D), jnp.float32),
            pltpu.VMEM_SHARED((N, D), jnp.float32),
        ],
    )
    def _scatter_sums(msg_hbm, sig_hbm, dst_hbm, zeros_hbm, ssh_hbm, ss_hbm,
                      idx_v, rows_v, accum):
        tid = lax.axis_index("s")
        core = lax.axis_index("c")

        pltpu.sync_copy(zeros_hbm.at[pl.ds(tid * NPT, NPT)],
                        accum.at[pl.ds(tid * NPT, NPT)])

        @pl.when(tid == NS - 1)
        def _():
            pltpu.sync_copy(zeros_hbm.at[pl.ds(NS * NPT, NTAIL)],
                            accum.at[pl.ds(NS * NPT, NTAIL)])

        plsc.subcore_barrier()

        def sweep(data_hbm):
            def step(i, _):
                off = tid * EPT + i * KS
                pltpu.sync_copy(dst_hbm.at[pl.ds(off, KS)], idx_v)
                pltpu.sync_copy(data_hbm.at[pl.ds(off, KS)], rows_v)
                pltpu.sync_copy(rows_v, accum.at[idx_v], add=True)
                return 0
            lax.fori_loop(0, _S_STEPS, step, 0)

        @pl.when(core == 0)
        def _():
            sweep(msg_hbm)

        @pl.when(core == 1)
        def _():
            sweep(sig_hbm)

        plsc.subcore_barrier()

        def drain(out_hbm):
            pltpu.sync_copy(accum.at[pl.ds(tid * NPT, NPT)],
                            out_hbm.at[pl.ds(tid * NPT, NPT)])

            @pl.when(tid == NS - 1)
            def _():
                pltpu.sync_copy(accum.at[pl.ds(NS * NPT, NTAIL)],
                                out_hbm.at[pl.ds(NS * NPT, NTAIL)])

        @pl.when(core == 0)
        def _():
            drain(ssh_hbm)

        @pl.when(core == 1)
        def _():
            drain(ss_hbm)

    return _scatter_sums


BE = 512


def _edge_body(ef_ref, ns_ref, nd_ref, wsg_ref, wdg_ref, weg_ref, wdu_ref,
               bm_ref, bdu_ref, lnw_ref, lnb_ref,
               y_ref, msg_ref, sig_ref):
    ef = ef_ref[...]
    ns = ns_ref[...]
    nd = nd_ref[...]
    m = (jnp.dot(ns, wsg_ref[...], preferred_element_type=jnp.float32)
         + jnp.dot(nd, wdg_ref[...], preferred_element_type=jnp.float32)
         + jnp.dot(ef, weg_ref[...], preferred_element_type=jnp.float32)
         + bm_ref[...])
    sig = jax.nn.sigmoid(m)
    bh = jnp.dot(ns, wdu_ref[...], preferred_element_type=jnp.float32) + bdu_ref[...]
    mu = jnp.mean(m, axis=-1, keepdims=True)
    var = jnp.mean((m - mu) * (m - mu), axis=-1, keepdims=True)
    mn = (m - mu) / jnp.sqrt(var + 1e-5) * lnw_ref[...] + lnb_ref[...]
    y_ref[...] = ef + jax.nn.silu(mn)
    msg_ref[...] = bh * sig
    sig_ref[...] = sig


def _edge_stage(ef, ns, nd, W_sg, W_dg, W_eg, W_du, b_m, b_du, ln_e_w, ln_e_b):
    row = pl.BlockSpec((BE, D), lambda i: (i, 0))
    full = pl.BlockSpec((D, D), lambda i: (0, 0))
    vec = pl.BlockSpec((1, D), lambda i: (0, 0))
    return pl.pallas_call(
        _edge_body,
        grid=(E // BE,),
        in_specs=[row, row, row, full, full, full, full, vec, vec, vec, vec],
        out_specs=[row, row, row],
        out_shape=[jax.ShapeDtypeStruct((E, D), jnp.float32)] * 3,
    )(ef, ns, nd, W_sg, W_dg, W_eg, W_du, b_m, b_du, ln_e_w, ln_e_b)


BN = 1000


def _node_body(nf_ref, ssh_ref, ss_ref, wsu_ref, bsu_ref, lnw_ref, lnb_ref,
               x_ref):
    nf = nf_ref[...]
    h = ssh_ref[...] / (ss_ref[...] + 1e-6)
    x = jnp.dot(nf, wsu_ref[...], preferred_element_type=jnp.float32) \
        + bsu_ref[...] + h
    mu = jnp.mean(x, axis=-1, keepdims=True)
    var = jnp.mean((x - mu) * (x - mu), axis=-1, keepdims=True)
    xn = (x - mu) / jnp.sqrt(var + 1e-5) * lnw_ref[...] + lnb_ref[...]
    x_ref[...] = nf + jax.nn.silu(xn)


def _node_stage(nf, ssh, ss, W_su, b_su, ln_n_w, ln_n_b):
    row = pl.BlockSpec((BN, D), lambda i: (i, 0))
    full = pl.BlockSpec((D, D), lambda i: (0, 0))
    vec = pl.BlockSpec((1, D), lambda i: (0, 0))
    return pl.pallas_call(
        _node_body,
        grid=(N // BN,),
        in_specs=[row, row, row, full, vec, vec, vec],
        out_specs=row,
        out_shape=jax.ShapeDtypeStruct((N, D), jnp.float32),
    )(nf, ssh, ss, W_su, b_su, ln_n_w, ln_n_b)


def kernel(node_feats, edge_feats, edge_index, W_sg, b_sg, W_dg, b_dg,
           W_eg, b_eg, W_su, b_su, W_du, b_du, ln_n_w, ln_n_b, ln_e_w, ln_e_b):
    src = edge_index[0].astype(jnp.int32)
    dst = edge_index[1].astype(jnp.int32)
    b_m = (b_sg + b_dg + b_eg).reshape(1, D)

    nf_src, nf_dst = _build_gather()(node_feats, src, dst)
    y, msg, sig = _edge_stage(
        edge_feats, nf_src, nf_dst, W_sg, W_dg, W_eg, W_du,
        b_m, b_du.reshape(1, D), ln_e_w.reshape(1, D), ln_e_b.reshape(1, D))
    zeros = jnp.zeros((N, D), jnp.float32)
    ssh, ss = _build_scatter()(msg, sig, dst, zeros)
    x = _node_stage(node_feats, ssh, ss, W_su, b_su.reshape(1, D),
                    ln_n_w.reshape(1, D), ln_n_b.reshape(1, D))
    return (x, y)


# speedup vs baseline: 3.7903x; 1.3867x over previous
import functools

import jax
import jax.numpy as jnp
from jax import lax
from jax.experimental import pallas as pl
from jax.experimental.pallas import tpu as pltpu
from jax.experimental.pallas import tpu_sc as plsc

N = 10000
E = 320000
D = 128

NC = 2
NS = 16
NW = NC * NS

NCHUNK = 5
CH = E // NCHUNK

EPW = CH // NW
KG = 80
_G_STEPS = EPW // KG
assert _G_STEPS * KG == EPW


def _sc_mesh():
    return plsc.VectorSubcoreMesh(core_axis_name="c", subcore_axis_name="s",
                                  num_cores=NC, num_subcores=NS)


@functools.cache
def _build_gather(coff):
    @functools.partial(
        pl.kernel,
        out_type=(
            jax.ShapeDtypeStruct((CH, D), jnp.float32),
            jax.ShapeDtypeStruct((CH, D), jnp.float32),
        ),
        mesh=_sc_mesh(),
        scratch_types=[
            pltpu.VMEM((KG,), jnp.int32),
            pltpu.VMEM((KG,), jnp.int32),
            pltpu.VMEM((KG, D), jnp.float32),
            pltpu.VMEM((KG, D), jnp.float32),
            pltpu.SemaphoreType.DMA,
            pltpu.SemaphoreType.DMA,
        ],
    )
    def _gather_nodes(nf_hbm, src_hbm, dst_hbm, osrc_hbm, odst_hbm,
                      idx_s, idx_d, rows_s, rows_d, sem_s, sem_d):
        wid = lax.axis_index("s") * NC + lax.axis_index("c")
        base = wid * EPW

        def step(i, _):
            off = base + i * KG
            pltpu.sync_copy(src_hbm.at[pl.ds(coff + off, KG)], idx_s)
            pltpu.sync_copy(dst_hbm.at[pl.ds(coff + off, KG)], idx_d)
            a = pltpu.async_copy(nf_hbm.at[idx_s], rows_s, sem_s)
            b = pltpu.async_copy(nf_hbm.at[idx_d], rows_d, sem_d)
            a.wait()
            b.wait()
            pltpu.sync_copy(rows_s, osrc_hbm.at[pl.ds(off, KG)])
            pltpu.sync_copy(rows_d, odst_hbm.at[pl.ds(off, KG)])
            return 0

        lax.fori_loop(0, _G_STEPS, step, 0)

    return _gather_nodes


EPT = CH // NS
KS = 80
_S_STEPS = EPT // KS
assert _S_STEPS * KS == EPT
NPT = (N // NS) // 8 * 8
NTAIL = N - NS * NPT


@functools.cache
def _build_scatter(coff):
    @functools.partial(
        pl.kernel,
        out_type=(
            jax.ShapeDtypeStruct((N, D), jnp.float32),
            jax.ShapeDtypeStruct((N, D), jnp.float32),
        ),
        mesh=_sc_mesh(),
        scratch_types=[
            pltpu.VMEM((KS,), jnp.int32),
            pltpu.VMEM((KS, D), jnp.float32),
            pltpu.VMEM_SHARED((N, D), jnp.float32),
        ],
    )
    def _scatter_sums(msg_hbm, sig_hbm, dst_hbm, zeros_hbm, ssh_hbm, ss_hbm,
                      idx_v, rows_v, accum):
        tid = lax.axis_index("s")
        core = lax.axis_index("c")

        pltpu.sync_copy(zeros_hbm.at[pl.ds(tid * NPT, NPT)],
                        accum.at[pl.ds(tid * NPT, NPT)])

        @pl.when(tid == NS - 1)
        def _():
            pltpu.sync_copy(zeros_hbm.at[pl.ds(NS * NPT, NTAIL)],
                            accum.at[pl.ds(NS * NPT, NTAIL)])

        plsc.subcore_barrier()

        def sweep(data_hbm):
            def step(i, _):
                off = tid * EPT + i * KS
                pltpu.sync_copy(dst_hbm.at[pl.ds(coff + off, KS)], idx_v)
                pltpu.sync_copy(data_hbm.at[pl.ds(off, KS)], rows_v)
                pltpu.sync_copy(rows_v, accum.at[idx_v], add=True)
                return 0
            lax.fori_loop(0, _S_STEPS, step, 0)

        @pl.when(core == 0)
        def _():
            sweep(msg_hbm)

        @pl.when(core == 1)
        def _():
            sweep(sig_hbm)

        plsc.subcore_barrier()

        def drain(out_hbm):
            pltpu.sync_copy(accum.at[pl.ds(tid * NPT, NPT)],
                            out_hbm.at[pl.ds(tid * NPT, NPT)])

            @pl.when(tid == NS - 1)
            def _():
                pltpu.sync_copy(accum.at[pl.ds(NS * NPT, NTAIL)],
                                out_hbm.at[pl.ds(NS * NPT, NTAIL)])

        @pl.when(core == 0)
        def _():
            drain(ssh_hbm)

        @pl.when(core == 1)
        def _():
            drain(ss_hbm)

    return _scatter_sums


BE = 512


def _edge_body(ef_ref, ns_ref, nd_ref, wsg_ref, wdg_ref, weg_ref, wdu_ref,
               bm_ref, bdu_ref, lnw_ref, lnb_ref, yin_ref,
               y_ref, msg_ref, sig_ref):
    del yin_ref
    ef = ef_ref[...]
    ns = ns_ref[...]
    nd = nd_ref[...]
    m = (jnp.dot(ns, wsg_ref[...], preferred_element_type=jnp.float32)
         + jnp.dot(nd, wdg_ref[...], preferred_element_type=jnp.float32)
         + jnp.dot(ef, weg_ref[...], preferred_element_type=jnp.float32)
         + bm_ref[...])
    sig = jax.nn.sigmoid(m)
    bh = jnp.dot(ns, wdu_ref[...], preferred_element_type=jnp.float32) + bdu_ref[...]
    mu = jnp.mean(m, axis=-1, keepdims=True)
    var = jnp.mean((m - mu) * (m - mu), axis=-1, keepdims=True)
    mn = (m - mu) / jnp.sqrt(var + 1e-5) * lnw_ref[...] + lnb_ref[...]
    y_ref[...] = ef + jax.nn.silu(mn)
    msg_ref[...] = bh * sig
    sig_ref[...] = sig


def _edge_stage(coff, ef, ns, nd, W_sg, W_dg, W_eg, W_du, b_m, b_du,
                ln_e_w, ln_e_b, y_prev):
    cblk = coff // BE
    rowc = pl.BlockSpec((BE, D), lambda i: (i, 0))
    rowf = pl.BlockSpec((BE, D), lambda i, cblk=cblk: (cblk + i, 0))
    full = pl.BlockSpec((D, D), lambda i: (0, 0))
    vec = pl.BlockSpec((1, D), lambda i: (0, 0))
    tiny = pl.BlockSpec((8, D), lambda i: (0, 0))
    return pl.pallas_call(
        _edge_body,
        grid=(CH // BE,),
        in_specs=[rowf, rowc, rowc, full, full, full, full, vec, vec, vec,
                  vec, tiny],
        out_specs=[rowf, rowc, rowc],
        out_shape=[
            jax.ShapeDtypeStruct((E, D), jnp.float32),
            jax.ShapeDtypeStruct((CH, D), jnp.float32),
            jax.ShapeDtypeStruct((CH, D), jnp.float32),
        ],
        input_output_aliases={11: 0},
    )(ef, ns, nd, W_sg, W_dg, W_eg, W_du, b_m, b_du, ln_e_w, ln_e_b, y_prev)


BN = 1000


def _node_body(nf_ref, *rest):
    ssh_refs = rest[:NCHUNK]
    ss_refs = rest[NCHUNK:2 * NCHUNK]
    wsu_ref, bsu_ref, lnw_ref, lnb_ref, x_ref = rest[2 * NCHUNK:]
    nf = nf_ref[...]
    ssh = ssh_refs[0][...]
    ss = ss_refs[0][...]
    for r in ssh_refs[1:]:
        ssh = ssh + r[...]
    for r in ss_refs[1:]:
        ss = ss + r[...]
    h = ssh / (ss + 1e-6)
    x = jnp.dot(nf, wsu_ref[...], preferred_element_type=jnp.float32) \
        + bsu_ref[...] + h
    mu = jnp.mean(x, axis=-1, keepdims=True)
    var = jnp.mean((x - mu) * (x - mu), axis=-1, keepdims=True)
    xn = (x - mu) / jnp.sqrt(var + 1e-5) * lnw_ref[...] + lnb_ref[...]
    x_ref[...] = nf + jax.nn.silu(xn)


def _node_stage(nf, sshs, sss, W_su, b_su, ln_n_w, ln_n_b):
    row = pl.BlockSpec((BN, D), lambda i: (i, 0))
    full = pl.BlockSpec((D, D), lambda i: (0, 0))
    vec = pl.BlockSpec((1, D), lambda i: (0, 0))
    return pl.pallas_call(
        _node_body,
        grid=(N // BN,),
        in_specs=[row] + [row] * (2 * NCHUNK) + [full, vec, vec, vec],
        out_specs=row,
        out_shape=jax.ShapeDtypeStruct((N, D), jnp.float32),
    )(nf, *sshs, *sss, W_su, b_su, ln_n_w, ln_n_b)


def kernel(node_feats, edge_feats, edge_index, W_sg, b_sg, W_dg, b_dg,
           W_eg, b_eg, W_su, b_su, W_du, b_du, ln_n_w, ln_n_b, ln_e_w, ln_e_b):
    src = edge_index[0].astype(jnp.int32)
    dst = edge_index[1].astype(jnp.int32)
    b_m = (b_sg + b_dg + b_eg).reshape(1, D)
    b_du2 = b_du.reshape(1, D)
    lnew = ln_e_w.reshape(1, D)
    lneb = ln_e_b.reshape(1, D)
    zeros = jnp.zeros((N, D), jnp.float32)

    y = jnp.zeros((E, D), jnp.float32)
    sshs, sss = [], []
    for c in range(NCHUNK):
        coff = c * CH
        ns_c, nd_c = _build_gather(coff)(node_feats, src, dst)
        y, msg_c, sig_c = _edge_stage(
            coff, edge_feats, ns_c, nd_c, W_sg, W_dg, W_eg, W_du,
            b_m, b_du2, lnew, lneb, y)
        ssh_c, ss_c = _build_scatter(coff)(msg_c, sig_c, dst, zeros)
        sshs.append(ssh_c)
        sss.append(ss_c)

    x = _node_stage(node_feats, sshs, sss, W_su, b_su.reshape(1, D),
                    ln_n_w.reshape(1, D), ln_n_b.reshape(1, D))
    return (x, y)


# speedup vs baseline: 4.1581x; 1.0970x over previous
import functools

import jax
import jax.numpy as jnp
from jax import lax
from jax.experimental import pallas as pl
from jax.experimental.pallas import tpu as pltpu
from jax.experimental.pallas import tpu_sc as plsc

N = 10000
E = 320000
D = 128

NC = 2
NS = 16
NW = NC * NS

NCHUNK = 5
CH = E // NCHUNK

EPW = CH // NW
KG = 80
_G_STEPS = EPW // KG
assert _G_STEPS * KG == EPW


def _sc_mesh():
    return plsc.VectorSubcoreMesh(core_axis_name="c", subcore_axis_name="s",
                                  num_cores=NC, num_subcores=NS)


@functools.cache
def _build_gather(coff):
    @functools.partial(
        pl.kernel,
        out_type=(
            jax.ShapeDtypeStruct((CH, D), jnp.float32),
            jax.ShapeDtypeStruct((CH, D), jnp.float32),
        ),
        mesh=_sc_mesh(),
        scratch_types=[
            pltpu.VMEM((KG,), jnp.int32),
            pltpu.VMEM((KG,), jnp.int32),
            pltpu.VMEM((KG,), jnp.int32),
            pltpu.VMEM((KG,), jnp.int32),
            pltpu.VMEM((KG, D), jnp.float32),
            pltpu.VMEM((KG, D), jnp.float32),
            pltpu.VMEM((KG, D), jnp.float32),
            pltpu.VMEM((KG, D), jnp.float32),
            pltpu.SemaphoreType.DMA,
            pltpu.SemaphoreType.DMA,
            pltpu.SemaphoreType.DMA,
            pltpu.SemaphoreType.DMA,
        ],
    )
    def _gather_nodes(nf_hbm, src_hbm, dst_hbm, osrc_hbm, odst_hbm,
                      ixs0, ixs1, ixd0, ixd1, rs0, rs1, rd0, rd1,
                      ss0, ss1, sd0, sd1):
        wid = lax.axis_index("s") * NC + lax.axis_index("c")
        base = wid * EPW
        ixs = (ixs0, ixs1)
        ixd = (ixd0, ixd1)
        rs = (rs0, rs1)
        rd = (rd0, rd1)
        sems_s = (ss0, ss1)
        sems_d = (sd0, sd1)

        pend = [None, None]
        for i in range(_G_STEPS):
            b = i % 2
            off = base + i * KG
            pltpu.sync_copy(src_hbm.at[pl.ds(coff + off, KG)], ixs[b])
            pltpu.sync_copy(dst_hbm.at[pl.ds(coff + off, KG)], ixd[b])
            ga = pltpu.async_copy(nf_hbm.at[ixs[b]], rs[b], sems_s[b])
            gb = pltpu.async_copy(nf_hbm.at[ixd[b]], rd[b], sems_d[b])
            if i >= 1:
                pa, pb, poff = pend[1 - b]
                pa.wait()
                pb.wait()
                pltpu.sync_copy(rs[1 - b], osrc_hbm.at[pl.ds(poff, KG)])
                pltpu.sync_copy(rd[1 - b], odst_hbm.at[pl.ds(poff, KG)])
            pend[b] = (ga, gb, off)
        b = (_G_STEPS - 1) % 2
        pa, pb, poff = pend[b]
        pa.wait()
        pb.wait()
        pltpu.sync_copy(rs[b], osrc_hbm.at[pl.ds(poff, KG)])
        pltpu.sync_copy(rd[b], odst_hbm.at[pl.ds(poff, KG)])

    return _gather_nodes


EPT = CH // NS
KS = 80
_S_STEPS = EPT // KS
assert _S_STEPS * KS == EPT
NPT = (N // NS) // 8 * 8
NTAIL = N - NS * NPT


@functools.cache
def _build_scatter(coff):
    @functools.partial(
        pl.kernel,
        out_type=(
            jax.ShapeDtypeStruct((N, D), jnp.float32),
            jax.ShapeDtypeStruct((N, D), jnp.float32),
        ),
        mesh=_sc_mesh(),
        scratch_types=[
            pltpu.VMEM((KS,), jnp.int32),
            pltpu.VMEM((KS,), jnp.int32),
            pltpu.VMEM((KS, D), jnp.float32),
            pltpu.VMEM((KS, D), jnp.float32),
            pltpu.VMEM_SHARED((N, D), jnp.float32),
            pltpu.SemaphoreType.DMA,
            pltpu.SemaphoreType.DMA,
            pltpu.SemaphoreType.DMA,
            pltpu.SemaphoreType.DMA,
        ],
    )
    def _scatter_sums(msg_hbm, sig_hbm, dst_hbm, zeros_hbm, ssh_hbm, ss_hbm,
                      ix0, ix1, rv0, rv1, accum, si0, si1, sc0, sc1):
        tid = lax.axis_index("s")
        core = lax.axis_index("c")
        ix = (ix0, ix1)
        rv = (rv0, rv1)
        sem_in = (si0, si1)
        sem_sc = (sc0, sc1)

        pltpu.sync_copy(zeros_hbm.at[pl.ds(tid * NPT, NPT)],
                        accum.at[pl.ds(tid * NPT, NPT)])

        @pl.when(tid == NS - 1)
        def _():
            pltpu.sync_copy(zeros_hbm.at[pl.ds(NS * NPT, NTAIL)],
                            accum.at[pl.ds(NS * NPT, NTAIL)])

        plsc.subcore_barrier()

        def sweep(data_hbm):
            pend = [None, None]
            for i in range(_S_STEPS):
                b = i % 2
                off = tid * EPT + i * KS
                if pend[b] is not None:
                    pend[b].wait()
                pltpu.sync_copy(dst_hbm.at[pl.ds(coff + off, KS)], ix[b])
                ld = pltpu.async_copy(data_hbm.at[pl.ds(off, KS)], rv[b],
                                      sem_in[b])
                ld.wait()
                pend[b] = pltpu.async_copy(rv[b], accum.at[ix[b]], sem_sc[b],
                                           add=True)
            for b in range(2):
                if pend[b] is not None:
                    pend[b].wait()

        @pl.when(core == 0)
        def _():
            sweep(msg_hbm)

        @pl.when(core == 1)
        def _():
            sweep(sig_hbm)

        plsc.subcore_barrier()

        def drain(out_hbm):
            pltpu.sync_copy(accum.at[pl.ds(tid * NPT, NPT)],
                            out_hbm.at[pl.ds(tid * NPT, NPT)])

            @pl.when(tid == NS - 1)
            def _():
                pltpu.sync_copy(accum.at[pl.ds(NS * NPT, NTAIL)],
                                out_hbm.at[pl.ds(NS * NPT, NTAIL)])

        @pl.when(core == 0)
        def _():
            drain(ssh_hbm)

        @pl.when(core == 1)
        def _():
            drain(ss_hbm)

    return _scatter_sums


BE = 512


def _edge_body(ef_ref, ns_ref, nd_ref, wsg_ref, wdg_ref, weg_ref, wdu_ref,
               bm_ref, bdu_ref, lnw_ref, lnb_ref, yin_ref,
               y_ref, msg_ref, sig_ref):
    del yin_ref
    ef = ef_ref[...]
    ns = ns_ref[...].astype(jnp.bfloat16)
    nd = nd_ref[...].astype(jnp.bfloat16)
    efb = ef.astype(jnp.bfloat16)
    m = (jnp.dot(ns, wsg_ref[...], preferred_element_type=jnp.float32)
         + jnp.dot(nd, wdg_ref[...], preferred_element_type=jnp.float32)
         + jnp.dot(efb, weg_ref[...], preferred_element_type=jnp.float32)
         + bm_ref[...])
    sig = jax.nn.sigmoid(m)
    bh = jnp.dot(ns, wdu_ref[...], preferred_element_type=jnp.float32) + bdu_ref[...]
    mu = jnp.mean(m, axis=-1, keepdims=True)
    var = jnp.mean((m - mu) * (m - mu), axis=-1, keepdims=True)
    mn = (m - mu) / jnp.sqrt(var + 1e-5) * lnw_ref[...] + lnb_ref[...]
    y_ref[...] = ef + jax.nn.silu(mn)
    msg_ref[...] = bh * sig
    sig_ref[...] = sig


def _edge_stage(coff, ef, ns, nd, W_sg, W_dg, W_eg, W_du, b_m, b_du,
                ln_e_w, ln_e_b, y_prev):
    cblk = coff // BE
    rowc = pl.BlockSpec((BE, D), lambda i: (i, 0))
    rowf = pl.BlockSpec((BE, D), lambda i, cblk=cblk: (cblk + i, 0))
    full = pl.BlockSpec((D, D), lambda i: (0, 0))
    vec = pl.BlockSpec((1, D), lambda i: (0, 0))
    tiny = pl.BlockSpec((8, D), lambda i: (0, 0))
    return pl.pallas_call(
        _edge_body,
        grid=(CH // BE,),
        in_specs=[rowf, rowc, rowc, full, full, full, full, vec, vec, vec,
                  vec, tiny],
        out_specs=[rowf, rowc, rowc],
        out_shape=[
            jax.ShapeDtypeStruct((E, D), jnp.float32),
            jax.ShapeDtypeStruct((CH, D), jnp.float32),
            jax.ShapeDtypeStruct((CH, D), jnp.float32),
        ],
        input_output_aliases={11: 0},
    )(ef, ns, nd, W_sg, W_dg, W_eg, W_du, b_m, b_du, ln_e_w, ln_e_b, y_prev)


BN = 1000


def _node_body(nf_ref, *rest):
    ssh_refs = rest[:NCHUNK]
    ss_refs = rest[NCHUNK:2 * NCHUNK]
    wsu_ref, bsu_ref, lnw_ref, lnb_ref, x_ref = rest[2 * NCHUNK:]
    nf = nf_ref[...]
    ssh = ssh_refs[0][...]
    ss = ss_refs[0][...]
    for r in ssh_refs[1:]:
        ssh = ssh + r[...]
    for r in ss_refs[1:]:
        ss = ss + r[...]
    h = ssh / (ss + 1e-6)
    x = jnp.dot(nf, wsu_ref[...], preferred_element_type=jnp.float32) \
        + bsu_ref[...] + h
    mu = jnp.mean(x, axis=-1, keepdims=True)
    var = jnp.mean((x - mu) * (x - mu), axis=-1, keepdims=True)
    xn = (x - mu) / jnp.sqrt(var + 1e-5) * lnw_ref[...] + lnb_ref[...]
    x_ref[...] = nf + jax.nn.silu(xn)


def _node_stage(nf, sshs, sss, W_su, b_su, ln_n_w, ln_n_b):
    row = pl.BlockSpec((BN, D), lambda i: (i, 0))
    full = pl.BlockSpec((D, D), lambda i: (0, 0))
    vec = pl.BlockSpec((1, D), lambda i: (0, 0))
    return pl.pallas_call(
        _node_body,
        grid=(N // BN,),
        in_specs=[row] + [row] * (2 * NCHUNK) + [full, vec, vec, vec],
        out_specs=row,
        out_shape=jax.ShapeDtypeStruct((N, D), jnp.float32),
    )(nf, *sshs, *sss, W_su, b_su, ln_n_w, ln_n_b)


def kernel(node_feats, edge_feats, edge_index, W_sg, b_sg, W_dg, b_dg,
           W_eg, b_eg, W_su, b_su, W_du, b_du, ln_n_w, ln_n_b, ln_e_w, ln_e_b):
    src = edge_index[0].astype(jnp.int32)
    dst = edge_index[1].astype(jnp.int32)
    b_m = (b_sg + b_dg + b_eg).reshape(1, D)
    b_du2 = b_du.reshape(1, D)
    lnew = ln_e_w.reshape(1, D)
    lneb = ln_e_b.reshape(1, D)
    bf = jnp.bfloat16
    wsg = W_sg.astype(bf)
    wdg = W_dg.astype(bf)
    weg = W_eg.astype(bf)
    wdu = W_du.astype(bf)
    zeros = jnp.zeros((N, D), jnp.float32)

    y = jnp.zeros((E, D), jnp.float32)
    sshs, sss = [], []
    for c in range(NCHUNK):
        coff = c * CH
        ns_c, nd_c = _build_gather(coff)(node_feats, src, dst)
        y, msg_c, sig_c = _edge_stage(
            coff, edge_feats, ns_c, nd_c, wsg, wdg, weg, wdu,
            b_m, b_du2, lnew, lneb, y)
        ssh_c, ss_c = _build_scatter(coff)(msg_c, sig_c, dst, zeros)
        sshs.append(ssh_c)
        sss.append(ss_c)

    x = _node_stage(node_feats, sshs, sss, W_su, b_su.reshape(1, D),
                    ln_n_w.reshape(1, D), ln_n_b.reshape(1, D))
    return (x, y)
